# Initial kernel scaffold; baseline (speedup 1.0000x reference)
import jax
import jax.numpy as jnp
from jax.experimental import pallas as pl


def kernel(color_stroke, alpha):
    raise NotImplementedError("write your pallas kernel here")


# speedup vs baseline: 18.6936x; 18.6936x over previous
import jax
import jax.numpy as jnp
from jax.experimental import pallas as pl
from jax.experimental.pallas import tpu as pltpu

_BS = 32


def _composite_kernel(alpha_ref, color_ref, out_ref, acc_ref, t_ref, k_ref,
                      *, ns):
    sb = pl.program_id(1)

    @pl.when(sb == 0)
    def _init():
        acc_ref[...] = jnp.zeros_like(acc_ref)
        t_ref[...] = jnp.ones_like(t_ref)
        k_ref[...] = jnp.zeros_like(k_ref)

    acc = acc_ref[...]
    t = t_ref[...]
    k = k_ref[...]

    def apply(s, acc, t, k, qualifying):
        a = alpha_ref[0, s]
        m = a > 0.1
        if not qualifying:
            m = jnp.logical_not(m)
        sel = jnp.logical_and(m, k < 10)
        af = jnp.where(sel, a, 0.0)
        ta = t * af
        acc = acc + ta[None, :, :] * color_ref[0, s]
        t = t - ta
        k = k + sel.astype(jnp.int32)
        return acc, t, k

    for s in reversed(range(_BS)):
        acc, t, k = apply(s, acc, t, k, True)

    @pl.when(sb == ns - 1)
    def _finish():
        a2, t2, k2 = acc, t, k
        for s in range(_BS):
            a2, t2, k2 = apply(s, a2, t2, k2, False)
        out_ref[0] = a2 + t2[None, :, :]

    acc_ref[...] = acc
    t_ref[...] = t
    k_ref[...] = k


def kernel(color_stroke, alpha):
    b, s, _, w, _ = color_stroke.shape
    ns = s // _BS
    alpha2 = alpha.reshape(b, s, w, w)

    grid = (b, ns)
    out = pl.pallas_call(
        lambda ar, cr, orf, accr, tr, kr: _composite_kernel(
            ar, cr, orf, accr, tr, kr, ns=ns),
        grid=grid,
        in_specs=[
            pl.BlockSpec((1, _BS, w, w), lambda bi, j: (bi, ns - 1 - j, 0, 0)),
            pl.BlockSpec((1, _BS, 3, w, w),
                         lambda bi, j: (bi, ns - 1 - j, 0, 0, 0)),
        ],
        out_specs=pl.BlockSpec((1, 3, w, w), lambda bi, j: (bi, 0, 0, 0)),
        out_shape=jax.ShapeDtypeStruct((b, 3, w, w), color_stroke.dtype),
        scratch_shapes=[
            pltpu.VMEM((3, w, w), jnp.float32),
            pltpu.VMEM((w, w), jnp.float32),
            pltpu.VMEM((w, w), jnp.int32),
        ],
        compiler_params=pltpu.CompilerParams(
            dimension_semantics=("arbitrary", "arbitrary")),
    )(alpha2, color_stroke)
    return out


# speedup vs baseline: 58.6954x; 3.1399x over previous
import jax
import jax.numpy as jnp
from jax.experimental import pallas as pl
from jax.experimental.pallas import tpu as pltpu

_BS = 16


def _composite_kernel(alpha_hbm, color_hbm, out_ref,
                      abuf, cbuf, acc_ref, t_ref, k_ref, scal,
                      sem_a, sem_c, *, ns, total):
    i = pl.program_id(0)
    j = jax.lax.rem(i, ns)
    slot = jax.lax.rem(i, 2)
    other = 1 - slot

    def start_copy(step, dst_slot):
        bb = step // ns
        src_j = (ns - 1) - jax.lax.rem(step, ns)
        pltpu.make_async_copy(
            alpha_hbm.at[bb, pl.ds(src_j * _BS, _BS)],
            abuf.at[dst_slot], sem_a.at[dst_slot]).start()
        pltpu.make_async_copy(
            color_hbm.at[bb, pl.ds(src_j * _BS, _BS)],
            cbuf.at[dst_slot], sem_c.at[dst_slot]).start()

    def wait_copy(dst_slot):
        pltpu.make_async_copy(
            alpha_hbm.at[0, pl.ds(0, _BS)],
            abuf.at[dst_slot], sem_a.at[dst_slot]).wait()
        pltpu.make_async_copy(
            color_hbm.at[0, pl.ds(0, _BS)],
            cbuf.at[dst_slot], sem_c.at[dst_slot]).wait()

    @pl.when(j == 0)
    def _new_image():
        acc_ref[...] = jnp.zeros_like(acc_ref)
        t_ref[...] = jnp.ones_like(t_ref)
        k_ref[...] = jnp.zeros_like(k_ref)
        scal[0] = 0

    @pl.when(i == 0)
    def _prologue():
        scal[1] = 1
        start_copy(0, 0)

    done_pre = scal[0]
    fetched = scal[1 + slot]

    def start_next():
        nxt = i + 1
        j_next = jax.lax.rem(nxt, ns)
        need = jnp.logical_or(j_next == 0, done_pre == 0)

        @pl.when(jnp.logical_and(nxt < total, need))
        def _():
            start_copy(nxt, 1 - slot)
        scal[1 + (1 - slot)] = jnp.where(nxt < total, need.astype(jnp.int32), 0)

    @pl.when(jnp.logical_or(j != ns - 1, done_pre == 1))
    def _early():
        start_next()

    def apply(a, c3, acc, t, k, qualifying):
        m = a > 0.1
        if not qualifying:
            m = jnp.logical_not(m)
        sel = jnp.logical_and(m, k < 10)
        af = jnp.where(sel, a, 0.0)
        ta = t * af
        acc = acc + ta[None, :, :] * c3
        t = t - ta
        k = k + sel.astype(jnp.int32)
        return acc, t, k

    @pl.when(fetched == 1)
    def _compute():
        wait_copy(slot)
        acc = acc_ref[...]
        t = t_ref[...]
        k = k_ref[...]
        for s in reversed(range(_BS)):
            acc, t, k = apply(abuf[slot, s], cbuf[slot, s], acc, t, k, True)
        acc_ref[...] = acc
        t_ref[...] = t
        k_ref[...] = k
        scal[0] = (jnp.min(k) >= 10).astype(jnp.int32)

    @pl.when(j == ns - 1)
    def _finish():
        @pl.when(scal[0] == 0)
        def _pass2():
            acc = acc_ref[...]
            t = t_ref[...]
            k = k_ref[...]
            for s in range(_BS):
                acc, t, k = apply(abuf[slot, s], cbuf[slot, s],
                                  acc, t, k, False)
            for s in range(_BS):
                acc, t, k = apply(abuf[other, s], cbuf[other, s],
                                  acc, t, k, False)
            acc_ref[...] = acc
            t_ref[...] = t

        out_ref[0] = acc_ref[...] + t_ref[...][None, :, :]

        @pl.when(done_pre == 0)
        def _late():
            start_next()


def kernel(color_stroke, alpha):
    b, s, _, w, _ = color_stroke.shape
    ns = s // _BS
    total = b * ns
    alpha2 = alpha.reshape(b, s, w, w)

    out = pl.pallas_call(
        lambda ar, cr, orf, *rest: _composite_kernel(
            ar, cr, orf, *rest, ns=ns, total=total),
        grid=(total,),
        in_specs=[
            pl.BlockSpec(memory_space=pl.ANY),
            pl.BlockSpec(memory_space=pl.ANY),
        ],
        out_specs=pl.BlockSpec((1, 3, w, w), lambda i: (i // ns, 0, 0, 0)),
        out_shape=jax.ShapeDtypeStruct((b, 3, w, w), color_stroke.dtype),
        scratch_shapes=[
            pltpu.VMEM((2, _BS, w, w), jnp.float32),
            pltpu.VMEM((2, _BS, 3, w, w), jnp.float32),
            pltpu.VMEM((3, w, w), jnp.float32),
            pltpu.VMEM((w, w), jnp.float32),
            pltpu.VMEM((w, w), jnp.int32),
            pltpu.SMEM((4,), jnp.int32),
            pltpu.SemaphoreType.DMA((2,)),
            pltpu.SemaphoreType.DMA((2,)),
        ],
        compiler_params=pltpu.CompilerParams(
            dimension_semantics=("arbitrary",)),
    )(alpha2, color_stroke)
    return out


# speedup vs baseline: 62.5734x; 1.0661x over previous
import jax
import jax.numpy as jnp
from jax.experimental import pallas as pl
from jax.experimental.pallas import tpu as pltpu

_BS = 8
_NSLOT = 3


def _composite_kernel(alpha_hbm, color_hbm, out_ref,
                      abuf, cbuf, acc_ref, t_ref, k_ref, scal,
                      sem_a, sem_c, *, ns, total):
    i = pl.program_id(0)
    j = jax.lax.rem(i, ns)
    slot = jax.lax.rem(i, _NSLOT)

    def start_copy(step, dst_slot):
        bb = step // ns
        src_j = (ns - 1) - jax.lax.rem(step, ns)
        pltpu.make_async_copy(
            alpha_hbm.at[bb, pl.ds(src_j * _BS, _BS)],
            abuf.at[dst_slot], sem_a.at[dst_slot]).start()
        pltpu.make_async_copy(
            color_hbm.at[bb, pl.ds(src_j * _BS, _BS)],
            cbuf.at[dst_slot], sem_c.at[dst_slot]).start()

    def wait_copy(dst_slot):
        pltpu.make_async_copy(
            alpha_hbm.at[0, pl.ds(0, _BS)],
            abuf.at[dst_slot], sem_a.at[dst_slot]).wait()
        pltpu.make_async_copy(
            color_hbm.at[0, pl.ds(0, _BS)],
            cbuf.at[dst_slot], sem_c.at[dst_slot]).wait()

    @pl.when(j == 0)
    def _new_image():
        acc_ref[...] = jnp.zeros_like(acc_ref)
        t_ref[...] = jnp.ones_like(t_ref)
        k_ref[...] = jnp.zeros_like(k_ref)
        scal[0] = 0

    @pl.when(i == 0)
    def _prologue():
        scal[1] = 1
        start_copy(0, 0)

    done_pre = scal[0]
    fetched = scal[1 + slot]

    def start_next():
        nxt = i + 1
        nslot = jax.lax.rem(nxt, _NSLOT)
        j_next = jax.lax.rem(nxt, ns)
        need = jnp.logical_or(j_next == 0, done_pre == 0)

        @pl.when(jnp.logical_and(nxt < total, need))
        def _():
            start_copy(nxt, nslot)
        scal[1 + nslot] = jnp.where(nxt < total, need.astype(jnp.int32), 0)

    @pl.when(jnp.logical_or(j != ns - 1, done_pre == 1))
    def _early():
        start_next()

    def apply(a, c3, acc, t, k, qualifying):
        m = a > 0.1
        if not qualifying:
            m = jnp.logical_not(m)
        sel = jnp.logical_and(m, k < 10)
        af = jnp.where(sel, a, 0.0)
        ta = t * af
        acc = acc + ta[None, :, :] * c3
        t = t - ta
        k = k + sel.astype(jnp.int32)
        return acc, t, k

    @pl.when(fetched == 1)
    def _compute():
        wait_copy(slot)
        acc = acc_ref[...]
        t = t_ref[...]
        k = k_ref[...]
        for s in reversed(range(_BS)):
            acc, t, k = apply(abuf[slot, s], cbuf[slot, s], acc, t, k, True)
        acc_ref[...] = acc
        t_ref[...] = t
        k_ref[...] = k
        scal[0] = (jnp.min(k) >= 10).astype(jnp.int32)

    @pl.when(j == ns - 1)
    def _finish():
        @pl.when(scal[0] == 0)
        def _pass2():
            acc = acc_ref[...]
            t = t_ref[...]
            k = k_ref[...]
            for p in range(_NSLOT):
                pslot = jax.lax.rem(i - p, _NSLOT)
                for s in range(_BS):
                    acc, t, k = apply(abuf[pslot, s], cbuf[pslot, s],
                                      acc, t, k, False)
            acc_ref[...] = acc
            t_ref[...] = t

        out_ref[0] = acc_ref[...] + t_ref[...][None, :, :]

        @pl.when(done_pre == 0)
        def _late():
            start_next()


def kernel(color_stroke, alpha):
    b, s, _, w, _ = color_stroke.shape
    ns = s // _BS
    total = b * ns
    alpha2 = alpha.reshape(b, s, w, w)

    out = pl.pallas_call(
        lambda ar, cr, orf, *rest: _composite_kernel(
            ar, cr, orf, *rest, ns=ns, total=total),
        grid=(total,),
        in_specs=[
            pl.BlockSpec(memory_space=pl.ANY),
            pl.BlockSpec(memory_space=pl.ANY),
        ],
        out_specs=pl.BlockSpec((1, 3, w, w), lambda i: (i // ns, 0, 0, 0)),
        out_shape=jax.ShapeDtypeStruct((b, 3, w, w), color_stroke.dtype),
        scratch_shapes=[
            pltpu.VMEM((_NSLOT, _BS, w, w), jnp.float32),
            pltpu.VMEM((_NSLOT, _BS, 3, w, w), jnp.float32),
            pltpu.VMEM((3, w, w), jnp.float32),
            pltpu.VMEM((w, w), jnp.float32),
            pltpu.VMEM((w, w), jnp.int32),
            pltpu.SMEM((1 + _NSLOT,), jnp.int32),
            pltpu.SemaphoreType.DMA((_NSLOT,)),
            pltpu.SemaphoreType.DMA((_NSLOT,)),
        ],
        compiler_params=pltpu.CompilerParams(
            dimension_semantics=("arbitrary",)),
    )(alpha2, color_stroke)
    return out


# speedup vs baseline: 90.4237x; 1.4451x over previous
import jax
import jax.numpy as jnp
from jax.experimental import pallas as pl
from jax.experimental.pallas import tpu as pltpu

_BS = 8
_NSLOT = 3


def _composite_kernel(alpha_hbm, color_hbm, out_ref,
                      abuf, cbuf, taubuf, acc_ref, t_ref, k_ref,
                      sem_a, sem_c, *, ns, nb):
    b = pl.program_id(0)

    def start_alpha(bb, jb, dst_slot):
        src_j = (ns - 1) - jb
        pltpu.make_async_copy(
            alpha_hbm.at[bb, pl.ds(src_j * _BS, _BS)],
            abuf.at[dst_slot], sem_a.at[dst_slot]).start()

    def start_color(bb, jb, dst_slot):
        src_j = (ns - 1) - jb
        pltpu.make_async_copy(
            color_hbm.at[bb, pl.ds(src_j * _BS, _BS)],
            cbuf.at[dst_slot], sem_c.at[dst_slot]).start()

    def wait_alpha(dst_slot):
        pltpu.make_async_copy(
            alpha_hbm.at[0, pl.ds(0, _BS)],
            abuf.at[dst_slot], sem_a.at[dst_slot]).wait()

    def wait_color(dst_slot):
        pltpu.make_async_copy(
            color_hbm.at[0, pl.ds(0, _BS)],
            cbuf.at[dst_slot], sem_c.at[dst_slot]).wait()

    acc_ref[...] = jnp.zeros_like(acc_ref)
    t_ref[...] = jnp.ones_like(t_ref)
    k_ref[...] = jnp.zeros_like(k_ref)

    @pl.when(b == 0)
    def _prologue():
        start_alpha(0, 0, 0)
        if ns > 1:
            start_alpha(0, 1, 1)
        start_color(0, 0, 0)

    def cond(carry):
        jb, done, _ = carry
        return jnp.logical_and(jb < ns, done == 0)

    def body(carry):
        jb, done, colp = carry
        slot = jax.lax.rem(jb, _NSLOT)
        wait_alpha(slot)

        @pl.when(jb + 2 < ns)
        def _prefetch_alpha():
            start_alpha(b, jb + 2, jax.lax.rem(jb + 2, _NSLOT))

        t = t_ref[...]
        k = k_ref[...]
        for s in reversed(range(_BS)):
            a = abuf[slot, s]
            sel = jnp.logical_and(a > 0.1, k < 10)
            ta = t * jnp.where(sel, a, 0.0)
            taubuf[slot, s] = ta
            t = t - ta
            k = k + sel.astype(jnp.int32)
        t_ref[...] = t
        k_ref[...] = k
        done_now = (jnp.min(k) >= 10).astype(jnp.int32)

        @pl.when(jnp.logical_and(jb + 1 < ns, done_now == 0))
        def _fetch_color():
            start_color(b, jb + 1, jax.lax.rem(jb + 1, _NSLOT))

        @pl.when(colp == 1)
        def _apply_color():
            wait_color(slot)
            acc = acc_ref[...]
            for s in range(_BS):
                acc = acc + taubuf[slot, s][None, :, :] * cbuf[slot, s]
            acc_ref[...] = acc

        new_colp = jnp.where(jb + 1 < ns, 1 - done_now, 0)
        return jb + 1, done_now, new_colp

    jb_exit, done_final, _ = jax.lax.while_loop(cond, body, (0, 0, 1))

    @pl.when(jb_exit < ns)
    def _drain0():
        wait_alpha(jax.lax.rem(jb_exit, _NSLOT))

    @pl.when(jb_exit + 1 < ns)
    def _drain1():
        wait_alpha(jax.lax.rem(jb_exit + 1, _NSLOT))

    @pl.when(done_final == 0)
    def _pass2():
        acc = acc_ref[...]
        t = t_ref[...]
        k = k_ref[...]
        for p in range(_NSLOT):
            pslot = (ns - 1 - p) % _NSLOT
            for s in range(_BS):
                a = abuf[pslot, s]
                sel = jnp.logical_and(a <= 0.1, k < 10)
                ta = t * jnp.where(sel, a, 0.0)
                acc = acc + ta[None, :, :] * cbuf[pslot, s]
                t = t - ta
                k = k + sel.astype(jnp.int32)
        acc_ref[...] = acc
        t_ref[...] = t

    out_ref[0] = acc_ref[...] + t_ref[...][None, :, :]

    @pl.when(b + 1 < nb)
    def _next_image():
        start_alpha(b + 1, 0, 0)
        if ns > 1:
            start_alpha(b + 1, 1, 1)
        start_color(b + 1, 0, 0)


def kernel(color_stroke, alpha):
    b, s, _, w, _ = color_stroke.shape
    ns = s // _BS
    alpha2 = alpha.reshape(b, s, w, w)

    out = pl.pallas_call(
        lambda ar, cr, orf, *rest: _composite_kernel(
            ar, cr, orf, *rest, ns=ns, nb=b),
        grid=(b,),
        in_specs=[
            pl.BlockSpec(memory_space=pl.ANY),
            pl.BlockSpec(memory_space=pl.ANY),
        ],
        out_specs=pl.BlockSpec((1, 3, w, w), lambda i: (i, 0, 0, 0)),
        out_shape=jax.ShapeDtypeStruct((b, 3, w, w), color_stroke.dtype),
        scratch_shapes=[
            pltpu.VMEM((_NSLOT, _BS, w, w), jnp.float32),
            pltpu.VMEM((_NSLOT, _BS, 3, w, w), jnp.float32),
            pltpu.VMEM((_NSLOT, _BS, w, w), jnp.float32),
            pltpu.VMEM((3, w, w), jnp.float32),
            pltpu.VMEM((w, w), jnp.float32),
            pltpu.VMEM((w, w), jnp.int32),
            pltpu.SemaphoreType.DMA((_NSLOT,)),
            pltpu.SemaphoreType.DMA((_NSLOT,)),
        ],
        compiler_params=pltpu.CompilerParams(
            dimension_semantics=("arbitrary",)),
    )(alpha2, color_stroke)
    return out
